# baseline (speedup 1.0000x reference)
import functools

import jax
import jax.numpy as jnp
from jax import lax
from jax.experimental import pallas as pl
from jax.experimental.pallas import tpu as pltpu
from jax.experimental.pallas import tpu_sc as plsc

T, D, F, E, TOPK = 2048, 1024, 2048, 8, 2
R = T * TOPK
BT = 128
BF = 512
NF = F // BF
NB = R // BT + E
ROWS = NB * BT


def _ffn_body(block_info_ref, a_ref, rw_ref, wg_ref, wu_ref, wd_ref, o_ref):
    i = pl.program_id(0)
    used = block_info_ref[i, 1]

    @pl.when(used == 0)
    def _zero():
        o_ref[...] = jnp.zeros_like(o_ref)

    @pl.when(used > 0)
    def _compute():
        a = a_ref[...]
        hg = jax.lax.dot_general(a, wg_ref[0], (((1,), (1,)), ((), ())),
                                 preferred_element_type=jnp.float32)
        hu = jax.lax.dot_general(a, wu_ref[0], (((1,), (1,)), ((), ())),
                                 preferred_element_type=jnp.float32)
        h = (hg * jax.nn.sigmoid(hg)) * hu
        y = jax.lax.dot_general(h, wd_ref[0], (((1,), (1,)), ((), ())),
                                preferred_element_type=jnp.float32)
        o_ref[...] = y * rw_ref[:, 0:1]


def _grouped_ffn(a_rows, row_w_wide, w_gate, w_up, w_down, block_info):
    return pl.pallas_call(
        _ffn_body,
        grid_spec=pltpu.PrefetchScalarGridSpec(
            num_scalar_prefetch=1,
            grid=(NB,),
            in_specs=[
                pl.BlockSpec((BT, D), lambda i, bi: (i, 0)),
                pl.BlockSpec((BT, 128), lambda i, bi: (i, 0)),
                pl.BlockSpec((1, F, D), lambda i, bi: (bi[i, 0], 0, 0)),
                pl.BlockSpec((1, F, D), lambda i, bi: (bi[i, 0], 0, 0)),
                pl.BlockSpec((1, D, F), lambda i, bi: (bi[i, 0], 0, 0)),
            ],
            out_specs=pl.BlockSpec((BT, D), lambda i, bi: (i, 0)),
        ),
        out_shape=jax.ShapeDtypeStruct((ROWS, D), jnp.float32),
        compiler_params=pltpu.CompilerParams(
            dimension_semantics=("arbitrary",),
        ),
    )(block_info, a_rows, row_w_wide, w_gate, w_up, w_down)


CTOK = 16
NWORK = 32
TPW = T // NWORK
NCH = TPW // CTOK


def _sc_combine(y_rows, pos):
    mesh = plsc.VectorSubcoreMesh(core_axis_name="c", subcore_axis_name="s")

    @functools.partial(
        pl.kernel, mesh=mesh,
        out_type=jax.ShapeDtypeStruct((T, D), jnp.float32),
        scratch_types=[
            pltpu.VMEM((2 * CTOK,), jnp.int32),
            pltpu.VMEM((2 * CTOK, D), jnp.float32),
            pltpu.VMEM((CTOK, D), jnp.float32),
            pltpu.SemaphoreType.DMA,
        ],
    )
    def k(y_hbm, pos_hbm, out_hbm, idx_v, rows_v, out_v, sem):
        wid = lax.axis_index("s") * 2 + lax.axis_index("c")
        base = wid * TPW

        def chunk(ch, _):
            t0 = base + ch * CTOK
            pltpu.sync_copy(pos_hbm.at[pl.ds(t0 * TOPK, TOPK * CTOK)], idx_v)
            pltpu.async_copy(y_hbm.at[idx_v], rows_v, sem).wait()

            def tok(j, _):
                for c in range(D // 16):
                    sl = pl.ds(c * 16, 16)
                    out_v[j, sl] = rows_v[2 * j, sl] + rows_v[2 * j + 1, sl]
                return 0

            lax.fori_loop(0, CTOK, tok, 0)
            pltpu.sync_copy(out_v, out_hbm.at[pl.ds(t0, CTOK)])
            return 0

        lax.fori_loop(0, NCH, chunk, 0)

    return k(y_rows, pos)


def kernel(hidden_states, gate_w, w_gate, w_up, w_down):
    x = hidden_states
    logits = x @ gate_w.T
    scores = jax.nn.softmax(logits, axis=-1)
    ecol = jnp.arange(E, dtype=jnp.int32)[None, :]
    i1 = jnp.argmax(scores, axis=-1).astype(jnp.int32)
    m1 = jnp.max(scores, axis=-1)
    masked = jnp.where(ecol == i1[:, None], -jnp.inf, scores)
    i2 = jnp.argmax(masked, axis=-1).astype(jnp.int32)
    m2 = jnp.max(masked, axis=-1)
    topk_idx = jnp.stack([i1, i2], axis=1)
    topk_w = jnp.stack([m1, m2], axis=1)
    topk_w = topk_w / jnp.sum(topk_w, axis=-1, keepdims=True)

    eid = topk_idx.reshape(-1).astype(jnp.int32)
    G, GW = 32, R // 32
    onehot = (eid[:, None] == jnp.arange(E)[None, :]).astype(jnp.int32)
    oh3 = onehot.reshape(G, GW, E)
    incl = jnp.cumsum(oh3, axis=1)
    gtot = incl[:, -1, :]
    goff = jnp.cumsum(gtot, axis=0) - gtot
    rank3 = incl - oh3 + goff[:, None, :]
    rank = jnp.take_along_axis(
        rank3.reshape(R, E), eid[:, None], axis=1)[:, 0]
    counts = jnp.sum(gtot, axis=0)
    padded = ((counts + BT - 1) // BT) * BT
    seg_start = (jnp.cumsum(padded) - padded).astype(jnp.int32)
    dest = seg_start[eid] + rank
    row_token = jnp.zeros(ROWS, jnp.int32).at[dest].set(
        jnp.arange(R, dtype=jnp.int32) // TOPK)
    blk = jnp.arange(NB, dtype=jnp.int32)
    blk_start = blk * BT
    seg_end = seg_start + padded
    blk_expert = jnp.sum(
        (blk_start[:, None] >= seg_end[None, :]).astype(jnp.int32), axis=1)
    blk_expert = jnp.minimum(blk_expert, E - 1)
    total_used = jnp.sum(padded).astype(jnp.int32)
    blk_used = (blk_start < total_used).astype(jnp.int32)
    block_info = jnp.stack([blk_expert, blk_used], axis=1)

    pos = dest
    row_w = jnp.zeros(ROWS, jnp.float32).at[dest].set(topk_w.reshape(-1))
    row_w_wide = jnp.broadcast_to(row_w[:, None], (ROWS, 128))

    a_rows = x[row_token]
    y_rows = _grouped_ffn(a_rows, row_w_wide,
                          w_gate.astype(jnp.bfloat16),
                          w_up.astype(jnp.bfloat16),
                          w_down.astype(jnp.bfloat16), block_info)
    out = _sc_combine(y_rows, pos)
    return out


# speedup vs baseline: 1.1899x; 1.1899x over previous
import functools

import jax
import jax.numpy as jnp
from jax import lax
from jax.experimental import pallas as pl
from jax.experimental.pallas import tpu as pltpu
from jax.experimental.pallas import tpu_sc as plsc

T, D, F, E, TOPK = 2048, 1024, 2048, 8, 2
R = T * TOPK
BT = 128
BF = 512
NF = F // BF
NB = R // BT + E
ROWS = NB * BT


def _ffn_body(block_info_ref, a_ref, rw_ref, wg_ref, wu_ref, wd_ref, o_ref):
    i = pl.program_id(0)
    used = block_info_ref[i, 1]

    @pl.when(used == 0)
    def _zero():
        o_ref[...] = jnp.zeros_like(o_ref)

    @pl.when(used > 0)
    def _compute():
        a = a_ref[...]
        hg = jax.lax.dot_general(a, wg_ref[0], (((1,), (1,)), ((), ())),
                                 preferred_element_type=jnp.float32)
        hu = jax.lax.dot_general(a, wu_ref[0], (((1,), (1,)), ((), ())),
                                 preferred_element_type=jnp.float32)
        h = (hg * jax.nn.sigmoid(hg)) * hu
        y = jax.lax.dot_general(h, wd_ref[0], (((1,), (1,)), ((), ())),
                                preferred_element_type=jnp.float32)
        o_ref[...] = y * rw_ref[:, 0:1]


def _grouped_ffn(a_rows, row_w_wide, w_gate, w_up, w_down, block_info):
    return pl.pallas_call(
        _ffn_body,
        grid_spec=pltpu.PrefetchScalarGridSpec(
            num_scalar_prefetch=1,
            grid=(NB,),
            in_specs=[
                pl.BlockSpec((BT, D), lambda i, bi: (i, 0)),
                pl.BlockSpec((BT, 128), lambda i, bi: (i, 0)),
                pl.BlockSpec((1, F, D), lambda i, bi: (bi[i, 0], 0, 0)),
                pl.BlockSpec((1, F, D), lambda i, bi: (bi[i, 0], 0, 0)),
                pl.BlockSpec((1, D, F), lambda i, bi: (bi[i, 0], 0, 0)),
            ],
            out_specs=pl.BlockSpec((BT, D), lambda i, bi: (i, 0)),
        ),
        out_shape=jax.ShapeDtypeStruct((ROWS, D), jnp.float32),
        compiler_params=pltpu.CompilerParams(
            dimension_semantics=("arbitrary",),
        ),
    )(block_info, a_rows, row_w_wide, w_gate, w_up, w_down)


CTOK = 32
NWORK = 32
TPW = T // NWORK
NCH = TPW // CTOK


def _sc_combine(y_rows, pos):
    mesh = plsc.VectorSubcoreMesh(core_axis_name="c", subcore_axis_name="s")

    @functools.partial(
        pl.kernel, mesh=mesh,
        out_type=jax.ShapeDtypeStruct((T, D), jnp.float32),
        scratch_types=[
            pltpu.VMEM((2 * CTOK,), jnp.int32),
            pltpu.VMEM((2 * CTOK, D), jnp.float32),
            pltpu.VMEM((CTOK, D), jnp.float32),
            pltpu.SemaphoreType.DMA,
        ],
    )
    def k(y_hbm, pos_hbm, out_hbm, idx_v, rows_v, out_v, sem):
        wid = lax.axis_index("s") * 2 + lax.axis_index("c")
        base = wid * TPW

        def chunk(ch, _):
            t0 = base + ch * CTOK
            pltpu.sync_copy(pos_hbm.at[pl.ds(t0 * TOPK, TOPK * CTOK)], idx_v)
            pltpu.async_copy(y_hbm.at[idx_v], rows_v, sem).wait()

            def tok(j, _):
                for c in range(D // 16):
                    sl = pl.ds(c * 16, 16)
                    out_v[j, sl] = rows_v[2 * j, sl] + rows_v[2 * j + 1, sl]
                return 0

            lax.fori_loop(0, CTOK, tok, 0)
            pltpu.sync_copy(out_v, out_hbm.at[pl.ds(t0, CTOK)])
            return 0

        lax.fori_loop(0, NCH, chunk, 0)

    return k(y_rows, pos)


def kernel(hidden_states, gate_w, w_gate, w_up, w_down):
    x = hidden_states
    logits = x @ gate_w.T
    scores = jax.nn.softmax(logits, axis=-1)
    ecol = jnp.arange(E, dtype=jnp.int32)[None, :]
    i1 = jnp.argmax(scores, axis=-1).astype(jnp.int32)
    m1 = jnp.max(scores, axis=-1)
    masked = jnp.where(ecol == i1[:, None], -jnp.inf, scores)
    i2 = jnp.argmax(masked, axis=-1).astype(jnp.int32)
    m2 = jnp.max(masked, axis=-1)
    topk_idx = jnp.stack([i1, i2], axis=1)
    topk_w = jnp.stack([m1, m2], axis=1)
    topk_w = topk_w / jnp.sum(topk_w, axis=-1, keepdims=True)

    eid = topk_idx.reshape(-1).astype(jnp.int32)
    G, GW = 32, R // 32
    onehot = (eid[:, None] == jnp.arange(E)[None, :]).astype(jnp.int32)
    oh3 = onehot.reshape(G, GW, E)
    incl = jnp.cumsum(oh3, axis=1)
    gtot = incl[:, -1, :]
    goff = jnp.cumsum(gtot, axis=0) - gtot
    rank3 = incl - oh3 + goff[:, None, :]
    rank = jnp.take_along_axis(
        rank3.reshape(R, E), eid[:, None], axis=1)[:, 0]
    counts = jnp.sum(gtot, axis=0)
    padded = ((counts + BT - 1) // BT) * BT
    seg_start = (jnp.cumsum(padded) - padded).astype(jnp.int32)
    dest = seg_start[eid] + rank
    row_token = jnp.zeros(ROWS, jnp.int32).at[dest].set(
        jnp.arange(R, dtype=jnp.int32) // TOPK)
    blk = jnp.arange(NB, dtype=jnp.int32)
    blk_start = blk * BT
    seg_end = seg_start + padded
    blk_expert = jnp.sum(
        (blk_start[:, None] >= seg_end[None, :]).astype(jnp.int32), axis=1)
    blk_expert = jnp.minimum(blk_expert, E - 1)
    total_used = jnp.sum(padded).astype(jnp.int32)
    blk_used = (blk_start < total_used).astype(jnp.int32)
    block_info = jnp.stack([blk_expert, blk_used], axis=1)

    pos = dest
    row_w = jnp.zeros(ROWS, jnp.float32).at[dest].set(topk_w.reshape(-1))
    row_w_wide = jnp.broadcast_to(row_w[:, None], (ROWS, 128))

    a_rows = x[row_token]
    y_rows = _grouped_ffn(a_rows, row_w_wide, w_gate, w_up, w_down, block_info)
    out = _sc_combine(y_rows, pos)
    return out


# speedup vs baseline: 1.5221x; 1.2792x over previous
import functools

import jax
import jax.numpy as jnp
from jax import lax
from jax.experimental import pallas as pl
from jax.experimental.pallas import tpu as pltpu
from jax.experimental.pallas import tpu_sc as plsc

T, D, F, E, TOPK = 2048, 1024, 2048, 8, 2
R = T * TOPK
BT = 256
BF = 512
NF = F // BF
NB = R // BT + E
ROWS = NB * BT


def _ffn_body(block_info_ref, a_ref, rw_ref, wg_ref, wu_ref, wd_ref, o_ref):
    i = pl.program_id(0)
    used = block_info_ref[i, 1]

    @pl.when(used == 0)
    def _zero():
        o_ref[...] = jnp.zeros_like(o_ref)

    @pl.when(used > 0)
    def _compute():
        a = a_ref[...]
        hg = jax.lax.dot_general(a, wg_ref[0], (((1,), (1,)), ((), ())),
                                 preferred_element_type=jnp.float32)
        hu = jax.lax.dot_general(a, wu_ref[0], (((1,), (1,)), ((), ())),
                                 preferred_element_type=jnp.float32)
        h = (hg * jax.nn.sigmoid(hg)) * hu
        y = jax.lax.dot_general(h, wd_ref[0], (((1,), (1,)), ((), ())),
                                preferred_element_type=jnp.float32)
        o_ref[...] = y * rw_ref[:, 0:1]


def _grouped_ffn(a_rows, row_w_wide, w_gate, w_up, w_down, block_info):
    return pl.pallas_call(
        _ffn_body,
        grid_spec=pltpu.PrefetchScalarGridSpec(
            num_scalar_prefetch=1,
            grid=(NB,),
            in_specs=[
                pl.BlockSpec((BT, D), lambda i, bi: (i, 0)),
                pl.BlockSpec((BT, 128), lambda i, bi: (i, 0)),
                pl.BlockSpec((1, F, D), lambda i, bi: (bi[i, 0], 0, 0)),
                pl.BlockSpec((1, F, D), lambda i, bi: (bi[i, 0], 0, 0)),
                pl.BlockSpec((1, D, F), lambda i, bi: (bi[i, 0], 0, 0)),
            ],
            out_specs=pl.BlockSpec((BT, D), lambda i, bi: (i, 0)),
        ),
        out_shape=jax.ShapeDtypeStruct((ROWS, D), jnp.float32),
        compiler_params=pltpu.CompilerParams(
            dimension_semantics=("arbitrary",),
        ),
    )(block_info, a_rows, row_w_wide, w_gate, w_up, w_down)


CTOK = 32
NWORK = 32
TPW = T // NWORK
NCH = TPW // CTOK


def _sc_combine(y_rows, pos):
    mesh = plsc.VectorSubcoreMesh(core_axis_name="c", subcore_axis_name="s")

    @functools.partial(
        pl.kernel, mesh=mesh,
        out_type=jax.ShapeDtypeStruct((T, D), jnp.float32),
        scratch_types=[
            pltpu.VMEM((2 * CTOK,), jnp.int32),
            pltpu.VMEM((2 * CTOK, D), jnp.float32),
            pltpu.VMEM((CTOK, D), jnp.float32),
            pltpu.SemaphoreType.DMA,
        ],
    )
    def k(y_hbm, pos_hbm, out_hbm, idx_v, rows_v, out_v, sem):
        wid = lax.axis_index("s") * 2 + lax.axis_index("c")
        base = wid * TPW

        def chunk(ch, _):
            t0 = base + ch * CTOK
            pltpu.sync_copy(pos_hbm.at[pl.ds(t0 * TOPK, TOPK * CTOK)], idx_v)
            pltpu.async_copy(y_hbm.at[idx_v], rows_v, sem).wait()

            def tok(j, _):
                for c in range(D // 16):
                    sl = pl.ds(c * 16, 16)
                    out_v[j, sl] = rows_v[2 * j, sl] + rows_v[2 * j + 1, sl]
                return 0

            lax.fori_loop(0, CTOK, tok, 0)
            pltpu.sync_copy(out_v, out_hbm.at[pl.ds(t0, CTOK)])
            return 0

        lax.fori_loop(0, NCH, chunk, 0)

    return k(y_rows, pos)


def kernel(hidden_states, gate_w, w_gate, w_up, w_down):
    x = hidden_states
    logits = x @ gate_w.T
    scores = jax.nn.softmax(logits, axis=-1)
    ecol = jnp.arange(E, dtype=jnp.int32)[None, :]
    i1 = jnp.argmax(scores, axis=-1).astype(jnp.int32)
    m1 = jnp.max(scores, axis=-1)
    masked = jnp.where(ecol == i1[:, None], -jnp.inf, scores)
    i2 = jnp.argmax(masked, axis=-1).astype(jnp.int32)
    m2 = jnp.max(masked, axis=-1)
    topk_idx = jnp.stack([i1, i2], axis=1)
    topk_w = jnp.stack([m1, m2], axis=1)
    topk_w = topk_w / jnp.sum(topk_w, axis=-1, keepdims=True)

    eid = topk_idx.reshape(-1).astype(jnp.int32)
    G, GW = 32, R // 32
    onehot = (eid[:, None] == jnp.arange(E)[None, :]).astype(jnp.int32)
    oh3 = onehot.reshape(G, GW, E)
    incl = jnp.cumsum(oh3, axis=1)
    gtot = incl[:, -1, :]
    goff = jnp.cumsum(gtot, axis=0) - gtot
    rank3 = incl - oh3 + goff[:, None, :]
    rank = jnp.take_along_axis(
        rank3.reshape(R, E), eid[:, None], axis=1)[:, 0]
    counts = jnp.sum(gtot, axis=0)
    padded = ((counts + BT - 1) // BT) * BT
    seg_start = (jnp.cumsum(padded) - padded).astype(jnp.int32)
    dest = seg_start[eid] + rank
    row_token = jnp.zeros(ROWS, jnp.int32).at[dest].set(
        jnp.arange(R, dtype=jnp.int32) // TOPK)
    blk = jnp.arange(NB, dtype=jnp.int32)
    blk_start = blk * BT
    seg_end = seg_start + padded
    blk_expert = jnp.sum(
        (blk_start[:, None] >= seg_end[None, :]).astype(jnp.int32), axis=1)
    blk_expert = jnp.minimum(blk_expert, E - 1)
    total_used = jnp.sum(padded).astype(jnp.int32)
    blk_used = (blk_start < total_used).astype(jnp.int32)
    block_info = jnp.stack([blk_expert, blk_used], axis=1)

    pos = dest
    row_w = jnp.zeros(ROWS, jnp.float32).at[dest].set(topk_w.reshape(-1))
    row_w_wide = jnp.broadcast_to(row_w[:, None], (ROWS, 128))

    a_rows = x[row_token]
    y_rows = _grouped_ffn(a_rows, row_w_wide, w_gate, w_up, w_down, block_info)
    out = _sc_combine(y_rows, pos)
    return out
